# baseline (speedup 1.0000x reference)
import functools

import jax
import jax.numpy as jnp
from jax import lax
from jax.experimental import pallas as pl
from jax.experimental.pallas import tpu as pltpu
from jax.experimental.pallas import tpu_sc as plsc

_N = 50000
_E = 800000
_NC = 2
_NS = 16
_NW = _NC * _NS
_NPAD = 51200
_SLICE = _NPAD // _NS
_ALPHA = 0.05
_SLOPE = 0.2

_f32 = jnp.float32
_i32 = jnp.int32


def _mesh():
    return plsc.VectorSubcoreMesh(
        core_axis_name="c", subcore_axis_name="s", num_cores=_NC, num_subcores=_NS
    )


def _make_phase_a(H, blend, CA=3200):
    NCH = _E // CA
    NV = CA // 16
    ZL = H * _SLICE

    out_type = [
        jax.ShapeDtypeStruct((H * _E,), _f32),
        jax.ShapeDtypeStruct((_NC * H * _NPAD,), _f32),
    ]
    if blend:
        out_type.append(jax.ShapeDtypeStruct((H * _E,), _f32))

    def _hbufs(n):
        return [pltpu.VMEM((CA,), _f32) for _ in range(n)]

    scratch = (
        [pltpu.VMEM((CA,), _i32) for _ in range(2)]
        + [pltpu.VMEM((CA,), _i32) for _ in range(2 * (H - 1))]
        + _hbufs(H)
        + _hbufs(H)
        + _hbufs(H)
        + _hbufs(H)
        + [pltpu.VMEM_SHARED((H * _NPAD,), _f32)]
        + (_hbufs(3 * H) if blend else [])
        + [pltpu.SemaphoreType.DMA, pltpu.SemaphoreType.DMA]
    )

    def body(*refs):
        n_in = 8 if blend else 6
        n_out = 3 if blend else 2
        ins, outs, scr = (refs[:n_in], refs[n_in:n_in + n_out],
                          list(refs[n_in + n_out:]))
        if blend:
            src_h, dst_h, el_h, er_h, ee_h, z_h, rn0_h, e0_h = ins
            eexp_o, esum_o, a0_o = outs
        else:
            src_h, dst_h, el_h, er_h, ee_h, z_h = ins
            eexp_o, esum_o = outs

        def take(n):
            out, scr[:n] = scr[:n], []
            return out

        src_v, dst_v = take(2)
        sd1 = take(2 * (H - 1))
        srch = [src_v] + sd1[0::2]
        dsth = [dst_v] + sd1[1::2]
        elb = take(H)
        erb = take(H)
        eeb = take(H)
        exb = take(H)
        (esum_s,) = take(1)
        if blend:
            rnb = take(H)
            e0b = take(H)
            a0b = take(H)
        sem_in, sem_out = scr[:2]

        c = lax.axis_index("c")
        s = lax.axis_index("s")
        wid = s * _NC + c

        pltpu.sync_copy(z_h, esum_s.at[pl.ds(s * ZL, ZL)])
        plsc.subcore_barrier()

        @pl.loop(wid, NCH, step=_NW)
        def _chunk(k):
            base = k * CA
            ds = [pltpu.async_copy(src_h.at[pl.ds(base, CA)], src_v, sem_in),
                  pltpu.async_copy(dst_h.at[pl.ds(base, CA)], dst_v, sem_in)]
            for h in range(H):
                ds.append(pltpu.async_copy(
                    ee_h.at[pl.ds(h * _E + base, CA)], eeb[h], sem_in))
                if blend:
                    ds.append(pltpu.async_copy(
                        e0_h.at[pl.ds(h * _E + base, CA)], e0b[h], sem_in))
            for d in ds:
                d.wait()

            if H > 1:
                @pl.loop(0, NV, unroll=2)
                def _idx(i):
                    off = i * 16
                    s16 = src_v[pl.ds(off, 16)]
                    d16 = dst_v[pl.ds(off, 16)]
                    for h in range(1, H):
                        srch[h][pl.ds(off, 16)] = s16 + h * _NPAD
                        dsth[h][pl.ds(off, 16)] = d16 + h * _NPAD

            for h in range(H):
                pltpu.sync_copy(el_h.at[srch[h]], elb[h])
                pltpu.sync_copy(er_h.at[dsth[h]], erb[h])
                if blend:
                    pltpu.sync_copy(rn0_h.at[dsth[h]], rnb[h])

            @pl.loop(0, NV, unroll=2)
            def _vec(i):
                off = i * 16
                sl16 = pl.ds(off, 16)
                for h in range(H):
                    e = elb[h][sl16] + erb[h][sl16] + eeb[h][sl16]
                    e = jnp.where(e >= 0.0, e, _SLOPE * e)
                    exb[h][sl16] = jnp.exp(e)
                    if blend:
                        a0b[h][sl16] = e0b[h][sl16] * rnb[h][sl16]

            os_ = []
            for h in range(H):
                pltpu.sync_copy(exb[h], esum_s.at[dsth[h]], add=True)
                os_.append(pltpu.async_copy(
                    exb[h], eexp_o.at[pl.ds(h * _E + base, CA)], sem_out))
                if blend:
                    os_.append(pltpu.async_copy(
                        a0b[h], a0_o.at[pl.ds(h * _E + base, CA)], sem_out))
            for d in os_:
                d.wait()

        plsc.subcore_barrier()
        pltpu.sync_copy(esum_s.at[pl.ds(s * ZL, ZL)],
                        esum_o.at[pl.ds(c * H * _NPAD + s * ZL, ZL)])

    return pl.kernel(body, out_type=tuple(out_type), mesh=_mesh(),
                     scratch_types=scratch)


def _make_fused0(CB=320):
    NCH = _E // CB

    out_type = (
        jax.ShapeDtypeStruct((_NC, _NPAD, 32), _f32),
        jax.ShapeDtypeStruct((2 * _E,), _f32),
        jax.ShapeDtypeStruct((_NC * _NPAD,), _f32),
    )
    scratch = (
        [pltpu.VMEM((CB,), _i32) for _ in range(4)]
        + [pltpu.VMEM((CB,), _f32) for _ in range(4)]
        + [pltpu.VMEM((CB, 32), _f32)]
        + [pltpu.VMEM_SHARED((_NPAD, 32), _f32),
           pltpu.VMEM_SHARED((_NPAD,), _f32),
           pltpu.SemaphoreType.DMA, pltpu.SemaphoreType.DMA]
    )

    def body(src_h, dst_h, el_h, er_h, ee_h, z32_h, za_h, feat_h,
             u_o, eexp_o, esum_o,
             src_v, dst_v, srcc_v, dstc_v, elb, erb, eeb, exv, F,
             u_s, es_s, sem, sem2):
        c = lax.axis_index("c")
        s = lax.axis_index("s")

        pltpu.sync_copy(z32_h, u_s.at[pl.ds(s * _SLICE, _SLICE)])
        pltpu.sync_copy(za_h, es_s.at[pl.ds(s * _SLICE, _SLICE)])
        plsc.subcore_barrier()

        @pl.loop(s, NCH, step=_NS)
        def _chunk(k):
            base = k * CB
            di = pltpu.async_copy(src_h.at[pl.ds(base, CB)], src_v, sem)
            d1 = pltpu.async_copy(dst_h.at[pl.ds(base, CB)], dst_v, sem2)
            d2 = pltpu.async_copy(ee_h.at[pl.ds(c * _E + base, CB)], eeb,
                                  sem2)
            di.wait()
            pltpu.sync_copy(feat_h.at[c].at[src_v], F)
            d1.wait()
            d2.wait()

            @pl.loop(0, CB // 16, unroll=2)
            def _cidx(i):
                off = i * 16
                srcc_v[pl.ds(off, 16)] = src_v[pl.ds(off, 16)] + c * _NPAD
                dstc_v[pl.ds(off, 16)] = dst_v[pl.ds(off, 16)] + c * _NPAD

            pltpu.sync_copy(el_h.at[srcc_v], elb)
            pltpu.sync_copy(er_h.at[dstc_v], erb)

            @pl.loop(0, CB // 16, unroll=2)
            def _vec(i):
                sl16 = pl.ds(i * 16, 16)
                e = elb[sl16] + erb[sl16] + eeb[sl16]
                e = jnp.where(e >= 0.0, e, _SLOPE * e)
                exv[sl16] = jnp.exp(e)

            @pl.loop(0, CB // 16)
            def _rows(rb):
                r0 = rb * 16
                wvec = exv[pl.ds(r0, 16)]
                for j in range(16):
                    wv = jnp.full((16,), wvec[j], _f32)
                    r = r0 + j
                    F[r, pl.ds(0, 16)] = F[r, pl.ds(0, 16)] * wv
                    F[r, pl.ds(16, 16)] = F[r, pl.ds(16, 16)] * wv

            pltpu.sync_copy(exv, es_s.at[dst_v], add=True)
            od = pltpu.async_copy(exv, eexp_o.at[pl.ds(c * _E + base, CB)],
                                  sem2)
            pltpu.sync_copy(F, u_s.at[dst_v], add=True)
            od.wait()

        plsc.subcore_barrier()
        sl = pl.ds(s * _SLICE, _SLICE)
        pltpu.sync_copy(u_s.at[sl], u_o.at[c].at[sl])
        pltpu.sync_copy(es_s.at[sl],
                        esum_o.at[pl.ds(c * _NPAD + s * _SLICE, _SLICE)])

    return pl.kernel(body, out_type=out_type, mesh=_mesh(),
                     scratch_types=scratch,
                     compiler_params=pltpu.CompilerParams(
                         use_tc_tiling_on_sc=False))


def _make_fused2(CB=1600):
    NCH = _E // CB

    out_type = (
        jax.ShapeDtypeStruct((_NC, _NPAD, 16), _f32),
        jax.ShapeDtypeStruct((_NC * _NPAD,), _f32),
    )
    scratch = (
        [pltpu.VMEM((CB,), _i32) for _ in range(2)]
        + [pltpu.VMEM((CB,), _f32) for _ in range(4)]
        + [pltpu.VMEM((CB, 16), _f32)]
        + [pltpu.VMEM_SHARED((_NPAD, 16), _f32),
           pltpu.VMEM_SHARED((_NPAD,), _f32),
           pltpu.SemaphoreType.DMA, pltpu.SemaphoreType.DMA]
    )

    def body(src_h, dst_h, el_h, er_h, ee_h, z16_h, za_h, feat_h,
             u_o, esum_o,
             src_v, dst_v, elb, erb, eeb, exv, F, u_s, es_s, sem, sem2):
        c = lax.axis_index("c")
        s = lax.axis_index("s")
        wid = s * _NC + c

        pltpu.sync_copy(z16_h, u_s.at[pl.ds(s * _SLICE, _SLICE)])
        pltpu.sync_copy(za_h, es_s.at[pl.ds(s * _SLICE, _SLICE)])
        plsc.subcore_barrier()

        @pl.loop(wid, NCH, step=_NW)
        def _chunk(k):
            base = k * CB
            di = pltpu.async_copy(src_h.at[pl.ds(base, CB)], src_v, sem)
            d1 = pltpu.async_copy(dst_h.at[pl.ds(base, CB)], dst_v, sem2)
            d2 = pltpu.async_copy(ee_h.at[pl.ds(base, CB)], eeb, sem2)
            di.wait()
            pltpu.sync_copy(feat_h.at[src_v], F)
            pltpu.sync_copy(el_h.at[src_v], elb)
            d1.wait()
            d2.wait()
            pltpu.sync_copy(er_h.at[dst_v], erb)

            @pl.loop(0, CB // 16, unroll=2)
            def _vec(i):
                sl16 = pl.ds(i * 16, 16)
                e = elb[sl16] + erb[sl16] + eeb[sl16]
                e = jnp.where(e >= 0.0, e, _SLOPE * e)
                exv[sl16] = jnp.exp(e)

            @pl.loop(0, CB // 16)
            def _rows(rb):
                r0 = rb * 16
                wvec = exv[pl.ds(r0, 16)]
                for j in range(16):
                    wv = jnp.full((16,), wvec[j], _f32)
                    r = r0 + j
                    F[r, pl.ds(0, 16)] = F[r, pl.ds(0, 16)] * wv

            pltpu.sync_copy(exv, es_s.at[dst_v], add=True)
            pltpu.sync_copy(F, u_s.at[dst_v], add=True)

        plsc.subcore_barrier()
        sl = pl.ds(s * _SLICE, _SLICE)
        pltpu.sync_copy(u_s.at[sl], u_o.at[c].at[sl])
        pltpu.sync_copy(es_s.at[sl],
                        esum_o.at[pl.ds(c * _NPAD + s * _SLICE, _SLICE)])

    return pl.kernel(body, out_type=out_type, mesh=_mesh(),
                     scratch_types=scratch,
                     compiler_params=pltpu.CompilerParams(
                         use_tc_tiling_on_sc=False))


def _make_phase_b01(blend, CB=640):
    NCH = _E // CB

    out_type = jax.ShapeDtypeStruct((_NC, _NPAD, 32), _f32)
    scratch = [
        pltpu.VMEM((CB,), _i32),
        pltpu.VMEM((CB,), _i32),
        pltpu.VMEM((CB,), _f32),
        pltpu.VMEM((CB, 32), _f32),
        pltpu.VMEM_SHARED((_NPAD, 32), _f32),
    ]
    if blend:
        scratch.append(pltpu.VMEM((CB,), _f32))
        scratch.append(pltpu.VMEM((CB,), _f32))
        scratch.append(pltpu.VMEM((CB,), _i32))
    scratch.append(pltpu.SemaphoreType.DMA)
    scratch.append(pltpu.SemaphoreType.DMA)

    def body(*refs):
        if blend:
            (src_h, dst_h, e1_h, a0_h, rn1_h, feat_h, z_h, u_o,
             src_v, dst_v, w_v, F, u_s, a0_v, rnb_v, dstc_v, sem,
             sem2) = refs
        else:
            (src_h, dst_h, e0_h, feat_h, z_h, u_o,
             src_v, dst_v, w_v, F, u_s, sem, sem2) = refs

        c = lax.axis_index("c")
        s = lax.axis_index("s")

        pltpu.sync_copy(z_h, u_s.at[pl.ds(s * _SLICE, _SLICE)])
        plsc.subcore_barrier()

        @pl.loop(s, NCH, step=_NS)
        def _chunk(k):
            base = k * CB
            di = pltpu.async_copy(src_h.at[pl.ds(base, CB)], src_v, sem)
            ds = [pltpu.async_copy(dst_h.at[pl.ds(base, CB)], dst_v, sem2)]
            if blend:
                ds.append(pltpu.async_copy(
                    e1_h.at[pl.ds(c * _E + base, CB)], w_v, sem2))
                ds.append(pltpu.async_copy(
                    a0_h.at[pl.ds(c * _E + base, CB)], a0_v, sem2))
            else:
                ds.append(pltpu.async_copy(
                    e0_h.at[pl.ds(c * _E + base, CB)], w_v, sem2))
            di.wait()
            pltpu.sync_copy(feat_h.at[c].at[src_v], F)
            for d in ds:
                d.wait()
            if blend:
                @pl.loop(0, CB // 16, unroll=2)
                def _cidx(i):
                    off = i * 16
                    dstc_v[pl.ds(off, 16)] = dst_v[pl.ds(off, 16)] + c * _NPAD

                pltpu.sync_copy(rn1_h.at[dstc_v], rnb_v)

                @pl.loop(0, CB // 16, unroll=2)
                def _wvec(i):
                    off = i * 16
                    w = ((1.0 - _ALPHA) * w_v[pl.ds(off, 16)]
                         * rnb_v[pl.ds(off, 16)]
                         + _ALPHA * a0_v[pl.ds(off, 16)])
                    w_v[pl.ds(off, 16)] = w

            @pl.loop(0, CB // 16)
            def _rows(rb):
                r0 = rb * 16
                wvec = w_v[pl.ds(r0, 16)]
                for j in range(16):
                    wv = jnp.full((16,), wvec[j], _f32)
                    r = r0 + j
                    F[r, pl.ds(0, 16)] = F[r, pl.ds(0, 16)] * wv
                    F[r, pl.ds(16, 16)] = F[r, pl.ds(16, 16)] * wv

            pltpu.sync_copy(F, u_s.at[dst_v], add=True)

        plsc.subcore_barrier()
        sl = pl.ds(s * _SLICE, _SLICE)
        pltpu.sync_copy(u_s.at[sl], u_o.at[c].at[sl])

    return pl.kernel(body, out_type=out_type, mesh=_mesh(),
                     scratch_types=scratch,
                     compiler_params=pltpu.CompilerParams(
                         use_tc_tiling_on_sc=False))


def _make_phase_b2(CB=1600):
    NCH = _E // CB

    out_type = jax.ShapeDtypeStruct((_NC, _NPAD, 16), _f32)
    scratch = [
        pltpu.VMEM((CB,), _i32),
        pltpu.VMEM((CB,), _i32),
        pltpu.VMEM((CB,), _f32),
        pltpu.VMEM((CB, 16), _f32),
        pltpu.VMEM_SHARED((_NPAD, 16), _f32),
    ]

    scratch.append(pltpu.SemaphoreType.DMA)
    scratch.append(pltpu.SemaphoreType.DMA)

    def body(src_h, dst_h, e2_h, feat_h, z_h, u_o, src_v, dst_v, w_v, F, u_s,
             sem, sem2):
        c = lax.axis_index("c")
        s = lax.axis_index("s")
        wid = s * _NC + c

        pltpu.sync_copy(z_h, u_s.at[pl.ds(s * _SLICE, _SLICE)])
        plsc.subcore_barrier()

        @pl.loop(wid, NCH, step=_NW)
        def _chunk(k):
            base = k * CB
            d1 = pltpu.async_copy(src_h.at[pl.ds(base, CB)], src_v, sem)
            d2 = pltpu.async_copy(dst_h.at[pl.ds(base, CB)], dst_v, sem2)
            d3 = pltpu.async_copy(e2_h.at[pl.ds(base, CB)], w_v, sem2)
            d1.wait()
            pltpu.sync_copy(feat_h.at[src_v], F)
            d2.wait()
            d3.wait()

            @pl.loop(0, CB // 16)
            def _rows(rb):
                r0 = rb * 16
                wvec = w_v[pl.ds(r0, 16)]
                for j in range(16):
                    wv = jnp.full((16,), wvec[j], _f32)
                    r = r0 + j
                    F[r, pl.ds(0, 16)] = F[r, pl.ds(0, 16)] * wv

            pltpu.sync_copy(F, u_s.at[dst_v], add=True)

        plsc.subcore_barrier()
        sl = pl.ds(s * _SLICE, _SLICE)
        pltpu.sync_copy(u_s.at[sl], u_o.at[c].at[sl])

    return pl.kernel(body, out_type=out_type, mesh=_mesh(),
                     scratch_types=scratch,
                     compiler_params=pltpu.CompilerParams(
                         use_tc_tiling_on_sc=False))


_fused_0 = _make_fused0()
_phase_a_1b = _make_phase_a(H=2, blend=True)
_phase_b_1 = _make_phase_b01(blend=True)
_fused_2 = _make_fused2()


def _ee_edges(emb, We, ae, H, e_feat):
    t = jnp.sum((emb @ We).reshape(8, H, 16) * ae[None], axis=-1)
    return t[e_feat].T.reshape(-1)


def _pad_nodes(arr):
    return jnp.pad(arr, ((0, _NPAD - _N),) + ((0, 0),) * (arr.ndim - 1))


def _hm_flat(arr):
    return jnp.pad(arr.T, ((0, 0), (0, _NPAD - _N))).reshape(-1)


def _rnorm(esum_parts, H):
    p = esum_parts.reshape(_NC, H, _NPAD)
    es = (p[0] + p[1])[:, :_N].T
    return jnp.where(es > 0.0, 1.0 / es, 0.0)


def kernel(x, edge_index, e_feat, fc_W, fc_b, emb0, W0, We0, al0, ar0, ae0,
           emb1, W1, We1, al1, ar1, ae1, emb2, W2, We2, al2, ar2, ae2, res2_W):
    src = edge_index[0]
    dst = edge_index[1]

    za2 = jnp.zeros((2 * _SLICE,), _f32)
    za1 = jnp.zeros((_SLICE,), _f32)
    z16 = jnp.zeros((_SLICE, 16), _f32)
    z32 = jnp.zeros((_SLICE, 32), _f32)

    h0 = x @ fc_W + fc_b
    feat0 = (h0 @ W0).reshape(_N, 2, 32)
    el0 = jnp.sum(feat0 * al0[None], -1)
    er0 = jnp.sum(feat0 * ar0[None], -1)
    feat0_hm = jnp.pad(feat0.transpose(1, 0, 2),
                       ((0, 0), (0, _NPAD - _N), (0, 0)))
    eeT0 = _ee_edges(emb0, We0, ae0, 2, e_feat)

    u0, eexp0, esum0 = _fused_0(src, dst, _hm_flat(el0), _hm_flat(er0),
                                eeT0, z32, za1, feat0_hm)
    es0 = esum0.reshape(_NC, _NPAD)[:, :_N].T
    rnorm0 = jnp.where(es0 > 0.0, 1.0 / es0, 0.0)

    rst0 = u0[:, :_N].transpose(1, 0, 2) * rnorm0[:, :, None]
    h1 = jax.nn.elu(rst0).reshape(_N, 64)
    feat1 = (h1 @ W1).reshape(_N, 2, 32)
    el1 = jnp.sum(feat1 * al1[None], -1)
    er1 = jnp.sum(feat1 * ar1[None], -1)
    feat1_hm = jnp.pad(feat1.transpose(1, 0, 2),
                       ((0, 0), (0, _NPAD - _N), (0, 0)))
    eeT1 = _ee_edges(emb1, We1, ae1, 2, e_feat)

    eexp1, esum1p, a0 = _phase_a_1b(src, dst, _hm_flat(el1),
                                    _hm_flat(er1), eeT1, za2,
                                    _hm_flat(rnorm0), eexp0)
    rn1_hm = _hm_flat(_rnorm(esum1p, 2))
    u1 = _phase_b_1(src, dst, eexp1, a0, rn1_hm, feat1_hm, z32)

    rst1 = u1[:, :_N].transpose(1, 0, 2) + h1.reshape(_N, 2, 32)
    h2 = jax.nn.elu(rst1).reshape(_N, 64)
    feat2 = h2 @ W2
    el2 = feat2 @ al2[0][:, None]
    er2 = feat2 @ ar2[0][:, None]
    feat2_p = _pad_nodes(feat2)
    eeT2 = _ee_edges(emb2, We2, ae2, 1, e_feat)
    resv2 = h2 @ res2_W

    u2, esum2 = _fused_2(src, dst, _hm_flat(el2), _hm_flat(er2),
                         eeT2, z16, za1, feat2_p)
    p2 = esum2.reshape(_NC, _NPAD)
    es2 = (p2[0] + p2[1])[:_N][:, None]
    rnorm2 = jnp.where(es2 > 0.0, 1.0 / es2, 0.0)

    logits = (u2[0, :_N] + u2[1, :_N]) * rnorm2 + resv2
    return logits


# speedup vs baseline: 1.0237x; 1.0237x over previous
import functools

import jax
import jax.numpy as jnp
from jax import lax
from jax.experimental import pallas as pl
from jax.experimental.pallas import tpu as pltpu
from jax.experimental.pallas import tpu_sc as plsc

_N = 50000
_E = 800000
_NC = 2
_NS = 16
_NW = _NC * _NS
_NPAD = 51200
_SLICE = _NPAD // _NS
_ALPHA = 0.05
_SLOPE = 0.2

_f32 = jnp.float32
_i32 = jnp.int32


def _mesh():
    return plsc.VectorSubcoreMesh(
        core_axis_name="c", subcore_axis_name="s", num_cores=_NC, num_subcores=_NS
    )


def _make_phase_a(H, blend, CA=3200):
    NCH = _E // CA
    NV = CA // 16
    ZL = H * _SLICE

    out_type = [
        jax.ShapeDtypeStruct((H * _E,), _f32),
        jax.ShapeDtypeStruct((_NC * H * _NPAD,), _f32),
    ]
    if blend:
        out_type.append(jax.ShapeDtypeStruct((H * _E,), _f32))

    def _hbufs(n):
        return [pltpu.VMEM((CA,), _f32) for _ in range(n)]

    scratch = (
        [pltpu.VMEM((CA,), _i32) for _ in range(2)]
        + [pltpu.VMEM((CA,), _i32) for _ in range(2 * (H - 1))]
        + _hbufs(H)
        + _hbufs(H)
        + _hbufs(H)
        + _hbufs(H)
        + [pltpu.VMEM_SHARED((H * _NPAD,), _f32)]
        + (_hbufs(3 * H) if blend else [])
        + [pltpu.SemaphoreType.DMA, pltpu.SemaphoreType.DMA]
    )

    def body(*refs):
        n_in = 8 if blend else 6
        n_out = 3 if blend else 2
        ins, outs, scr = (refs[:n_in], refs[n_in:n_in + n_out],
                          list(refs[n_in + n_out:]))
        if blend:
            src_h, dst_h, el_h, er_h, ee_h, z_h, rn0_h, e0_h = ins
            eexp_o, esum_o, a0_o = outs
        else:
            src_h, dst_h, el_h, er_h, ee_h, z_h = ins
            eexp_o, esum_o = outs

        def take(n):
            out, scr[:n] = scr[:n], []
            return out

        src_v, dst_v = take(2)
        sd1 = take(2 * (H - 1))
        srch = [src_v] + sd1[0::2]
        dsth = [dst_v] + sd1[1::2]
        elb = take(H)
        erb = take(H)
        eeb = take(H)
        exb = take(H)
        (esum_s,) = take(1)
        if blend:
            rnb = take(H)
            e0b = take(H)
            a0b = take(H)
        sem_in, sem_out = scr[:2]

        c = lax.axis_index("c")
        s = lax.axis_index("s")
        wid = s * _NC + c

        pltpu.sync_copy(z_h, esum_s.at[pl.ds(s * ZL, ZL)])
        plsc.subcore_barrier()

        @pl.loop(wid, NCH, step=_NW)
        def _chunk(k):
            base = k * CA
            ds = [pltpu.async_copy(src_h.at[pl.ds(base, CA)], src_v, sem_in),
                  pltpu.async_copy(dst_h.at[pl.ds(base, CA)], dst_v, sem_in)]
            for h in range(H):
                ds.append(pltpu.async_copy(
                    ee_h.at[pl.ds(h * _E + base, CA)], eeb[h], sem_in))
                if blend:
                    ds.append(pltpu.async_copy(
                        e0_h.at[pl.ds(h * _E + base, CA)], e0b[h], sem_in))
            for d in ds:
                d.wait()

            if H > 1:
                @pl.loop(0, NV, unroll=2)
                def _idx(i):
                    off = i * 16
                    s16 = src_v[pl.ds(off, 16)]
                    d16 = dst_v[pl.ds(off, 16)]
                    for h in range(1, H):
                        srch[h][pl.ds(off, 16)] = s16 + h * _NPAD
                        dsth[h][pl.ds(off, 16)] = d16 + h * _NPAD

            for h in range(H):
                pltpu.sync_copy(el_h.at[srch[h]], elb[h])
                pltpu.sync_copy(er_h.at[dsth[h]], erb[h])
                if blend:
                    pltpu.sync_copy(rn0_h.at[dsth[h]], rnb[h])

            @pl.loop(0, NV, unroll=2)
            def _vec(i):
                off = i * 16
                sl16 = pl.ds(off, 16)
                for h in range(H):
                    e = elb[h][sl16] + erb[h][sl16] + eeb[h][sl16]
                    e = jnp.where(e >= 0.0, e, _SLOPE * e)
                    exb[h][sl16] = jnp.exp(e)
                    if blend:
                        a0b[h][sl16] = e0b[h][sl16] * rnb[h][sl16]

            os_ = []
            for h in range(H):
                pltpu.sync_copy(exb[h], esum_s.at[dsth[h]], add=True)
                os_.append(pltpu.async_copy(
                    exb[h], eexp_o.at[pl.ds(h * _E + base, CA)], sem_out))
                if blend:
                    os_.append(pltpu.async_copy(
                        a0b[h], a0_o.at[pl.ds(h * _E + base, CA)], sem_out))
            for d in os_:
                d.wait()

        plsc.subcore_barrier()
        pltpu.sync_copy(esum_s.at[pl.ds(s * ZL, ZL)],
                        esum_o.at[pl.ds(c * H * _NPAD + s * ZL, ZL)])

    return pl.kernel(body, out_type=tuple(out_type), mesh=_mesh(),
                     scratch_types=scratch)


def _make_fused0(CB=640, NR=50048):
    NCH = _E // CB
    SLR = NR // _NS

    out_type = (
        jax.ShapeDtypeStruct((_NC, NR, 32), _f32),
        jax.ShapeDtypeStruct((2 * _E,), _f32),
        jax.ShapeDtypeStruct((_NC * _NPAD,), _f32),
    )
    scratch = (
        [pltpu.VMEM((CB,), _i32) for _ in range(3)]
        + [pltpu.VMEM((CB,), _f32) for _ in range(4)]
        + [pltpu.VMEM((CB, 32), _f32)]
        + [pltpu.VMEM_SHARED((NR, 32), _f32),
           pltpu.VMEM_SHARED((_NPAD,), _f32),
           pltpu.SemaphoreType.DMA, pltpu.SemaphoreType.DMA]
    )

    def body(src_h, dst_h, el_h, er_h, ee_h, z32_h, za_h, feat_h,
             u_o, eexp_o, esum_o,
             src_v, dst_v, dstc_v, elb, erb, eeb, exv, F,
             u_s, es_s, sem, sem2):
        c = lax.axis_index("c")
        s = lax.axis_index("s")

        pltpu.sync_copy(z32_h, u_s.at[pl.ds(s * SLR, SLR)])
        pltpu.sync_copy(za_h, es_s.at[pl.ds(s * _SLICE, _SLICE)])
        plsc.subcore_barrier()

        @pl.loop(s, NCH, step=_NS)
        def _chunk(k):
            base = k * CB
            di = pltpu.async_copy(src_h.at[pl.ds(base, CB)], src_v, sem)
            d1 = pltpu.async_copy(dst_h.at[pl.ds(base, CB)], dst_v, sem2)
            d2 = pltpu.async_copy(ee_h.at[pl.ds(c * _E + base, CB)], eeb,
                                  sem2)
            di.wait()
            pltpu.sync_copy(feat_h.at[c].at[src_v], F)
            d1.wait()
            d2.wait()

            @pl.loop(0, CB // 16, unroll=2)
            def _cidx(i):
                off = i * 16
                src_v[pl.ds(off, 16)] = src_v[pl.ds(off, 16)] + c * _NPAD
                dstc_v[pl.ds(off, 16)] = dst_v[pl.ds(off, 16)] + c * _NPAD

            pltpu.sync_copy(el_h.at[src_v], elb)
            pltpu.sync_copy(er_h.at[dstc_v], erb)

            @pl.loop(0, CB // 16, unroll=2)
            def _vec(i):
                sl16 = pl.ds(i * 16, 16)
                e = elb[sl16] + erb[sl16] + eeb[sl16]
                e = jnp.where(e >= 0.0, e, _SLOPE * e)
                exv[sl16] = jnp.exp(e)

            @pl.loop(0, CB // 16)
            def _rows(rb):
                r0 = rb * 16
                wvec = exv[pl.ds(r0, 16)]
                for j in range(16):
                    wv = jnp.full((16,), wvec[j], _f32)
                    r = r0 + j
                    F[r, pl.ds(0, 16)] = F[r, pl.ds(0, 16)] * wv
                    F[r, pl.ds(16, 16)] = F[r, pl.ds(16, 16)] * wv

            pltpu.sync_copy(exv, es_s.at[dst_v], add=True)
            od = pltpu.async_copy(exv, eexp_o.at[pl.ds(c * _E + base, CB)],
                                  sem2)
            pltpu.sync_copy(F, u_s.at[dst_v], add=True)
            od.wait()

        plsc.subcore_barrier()
        slr = pl.ds(s * SLR, SLR)
        pltpu.sync_copy(u_s.at[slr], u_o.at[c].at[slr])
        sl = pl.ds(s * _SLICE, _SLICE)
        pltpu.sync_copy(es_s.at[sl],
                        esum_o.at[pl.ds(c * _NPAD + s * _SLICE, _SLICE)])

    return pl.kernel(body, out_type=out_type, mesh=_mesh(),
                     scratch_types=scratch,
                     compiler_params=pltpu.CompilerParams(
                         use_tc_tiling_on_sc=False))


def _make_fused2(CB=1600):
    NCH = _E // CB

    out_type = (
        jax.ShapeDtypeStruct((_NC, _NPAD, 16), _f32),
        jax.ShapeDtypeStruct((_NC * _NPAD,), _f32),
    )
    scratch = (
        [pltpu.VMEM((CB,), _i32) for _ in range(2)]
        + [pltpu.VMEM((CB,), _f32) for _ in range(4)]
        + [pltpu.VMEM((CB, 16), _f32)]
        + [pltpu.VMEM_SHARED((_NPAD, 16), _f32),
           pltpu.VMEM_SHARED((_NPAD,), _f32),
           pltpu.SemaphoreType.DMA, pltpu.SemaphoreType.DMA]
    )

    def body(src_h, dst_h, el_h, er_h, ee_h, z16_h, za_h, feat_h,
             u_o, esum_o,
             src_v, dst_v, elb, erb, eeb, exv, F, u_s, es_s, sem, sem2):
        c = lax.axis_index("c")
        s = lax.axis_index("s")
        wid = s * _NC + c

        pltpu.sync_copy(z16_h, u_s.at[pl.ds(s * _SLICE, _SLICE)])
        pltpu.sync_copy(za_h, es_s.at[pl.ds(s * _SLICE, _SLICE)])
        plsc.subcore_barrier()

        @pl.loop(wid, NCH, step=_NW)
        def _chunk(k):
            base = k * CB
            di = pltpu.async_copy(src_h.at[pl.ds(base, CB)], src_v, sem)
            d1 = pltpu.async_copy(dst_h.at[pl.ds(base, CB)], dst_v, sem2)
            d2 = pltpu.async_copy(ee_h.at[pl.ds(base, CB)], eeb, sem2)
            di.wait()
            pltpu.sync_copy(feat_h.at[src_v], F)
            pltpu.sync_copy(el_h.at[src_v], elb)
            d1.wait()
            d2.wait()
            pltpu.sync_copy(er_h.at[dst_v], erb)

            @pl.loop(0, CB // 16, unroll=2)
            def _vec(i):
                sl16 = pl.ds(i * 16, 16)
                e = elb[sl16] + erb[sl16] + eeb[sl16]
                e = jnp.where(e >= 0.0, e, _SLOPE * e)
                exv[sl16] = jnp.exp(e)

            @pl.loop(0, CB // 16)
            def _rows(rb):
                r0 = rb * 16
                wvec = exv[pl.ds(r0, 16)]
                for j in range(16):
                    wv = jnp.full((16,), wvec[j], _f32)
                    r = r0 + j
                    F[r, pl.ds(0, 16)] = F[r, pl.ds(0, 16)] * wv

            pltpu.sync_copy(exv, es_s.at[dst_v], add=True)
            pltpu.sync_copy(F, u_s.at[dst_v], add=True)

        plsc.subcore_barrier()
        sl = pl.ds(s * _SLICE, _SLICE)
        pltpu.sync_copy(u_s.at[sl], u_o.at[c].at[sl])
        pltpu.sync_copy(es_s.at[sl],
                        esum_o.at[pl.ds(c * _NPAD + s * _SLICE, _SLICE)])

    return pl.kernel(body, out_type=out_type, mesh=_mesh(),
                     scratch_types=scratch,
                     compiler_params=pltpu.CompilerParams(
                         use_tc_tiling_on_sc=False))


def _make_phase_b01(blend, CB=640):
    NCH = _E // CB

    out_type = jax.ShapeDtypeStruct((_NC, _NPAD, 32), _f32)
    scratch = [
        pltpu.VMEM((CB,), _i32),
        pltpu.VMEM((CB,), _i32),
        pltpu.VMEM((CB,), _f32),
        pltpu.VMEM((CB, 32), _f32),
        pltpu.VMEM_SHARED((_NPAD, 32), _f32),
    ]
    if blend:
        scratch.append(pltpu.VMEM((CB,), _f32))
        scratch.append(pltpu.VMEM((CB,), _f32))
        scratch.append(pltpu.VMEM((CB,), _i32))
    scratch.append(pltpu.SemaphoreType.DMA)
    scratch.append(pltpu.SemaphoreType.DMA)

    def body(*refs):
        if blend:
            (src_h, dst_h, e1_h, a0_h, rn1_h, feat_h, z_h, u_o,
             src_v, dst_v, w_v, F, u_s, a0_v, rnb_v, dstc_v, sem,
             sem2) = refs
        else:
            (src_h, dst_h, e0_h, feat_h, z_h, u_o,
             src_v, dst_v, w_v, F, u_s, sem, sem2) = refs

        c = lax.axis_index("c")
        s = lax.axis_index("s")

        pltpu.sync_copy(z_h, u_s.at[pl.ds(s * _SLICE, _SLICE)])
        plsc.subcore_barrier()

        @pl.loop(s, NCH, step=_NS)
        def _chunk(k):
            base = k * CB
            di = pltpu.async_copy(src_h.at[pl.ds(base, CB)], src_v, sem)
            ds = [pltpu.async_copy(dst_h.at[pl.ds(base, CB)], dst_v, sem2)]
            if blend:
                ds.append(pltpu.async_copy(
                    e1_h.at[pl.ds(c * _E + base, CB)], w_v, sem2))
                ds.append(pltpu.async_copy(
                    a0_h.at[pl.ds(c * _E + base, CB)], a0_v, sem2))
            else:
                ds.append(pltpu.async_copy(
                    e0_h.at[pl.ds(c * _E + base, CB)], w_v, sem2))
            di.wait()
            pltpu.sync_copy(feat_h.at[c].at[src_v], F)
            for d in ds:
                d.wait()
            if blend:
                @pl.loop(0, CB // 16, unroll=2)
                def _cidx(i):
                    off = i * 16
                    dstc_v[pl.ds(off, 16)] = dst_v[pl.ds(off, 16)] + c * _NPAD

                pltpu.sync_copy(rn1_h.at[dstc_v], rnb_v)

                @pl.loop(0, CB // 16, unroll=2)
                def _wvec(i):
                    off = i * 16
                    w = ((1.0 - _ALPHA) * w_v[pl.ds(off, 16)]
                         * rnb_v[pl.ds(off, 16)]
                         + _ALPHA * a0_v[pl.ds(off, 16)])
                    w_v[pl.ds(off, 16)] = w

            @pl.loop(0, CB // 16)
            def _rows(rb):
                r0 = rb * 16
                wvec = w_v[pl.ds(r0, 16)]
                for j in range(16):
                    wv = jnp.full((16,), wvec[j], _f32)
                    r = r0 + j
                    F[r, pl.ds(0, 16)] = F[r, pl.ds(0, 16)] * wv
                    F[r, pl.ds(16, 16)] = F[r, pl.ds(16, 16)] * wv

            pltpu.sync_copy(F, u_s.at[dst_v], add=True)

        plsc.subcore_barrier()
        sl = pl.ds(s * _SLICE, _SLICE)
        pltpu.sync_copy(u_s.at[sl], u_o.at[c].at[sl])

    return pl.kernel(body, out_type=out_type, mesh=_mesh(),
                     scratch_types=scratch,
                     compiler_params=pltpu.CompilerParams(
                         use_tc_tiling_on_sc=False))


def _make_phase_b2(CB=1600):
    NCH = _E // CB

    out_type = jax.ShapeDtypeStruct((_NC, _NPAD, 16), _f32)
    scratch = [
        pltpu.VMEM((CB,), _i32),
        pltpu.VMEM((CB,), _i32),
        pltpu.VMEM((CB,), _f32),
        pltpu.VMEM((CB, 16), _f32),
        pltpu.VMEM_SHARED((_NPAD, 16), _f32),
    ]

    scratch.append(pltpu.SemaphoreType.DMA)
    scratch.append(pltpu.SemaphoreType.DMA)

    def body(src_h, dst_h, e2_h, feat_h, z_h, u_o, src_v, dst_v, w_v, F, u_s,
             sem, sem2):
        c = lax.axis_index("c")
        s = lax.axis_index("s")
        wid = s * _NC + c

        pltpu.sync_copy(z_h, u_s.at[pl.ds(s * _SLICE, _SLICE)])
        plsc.subcore_barrier()

        @pl.loop(wid, NCH, step=_NW)
        def _chunk(k):
            base = k * CB
            d1 = pltpu.async_copy(src_h.at[pl.ds(base, CB)], src_v, sem)
            d2 = pltpu.async_copy(dst_h.at[pl.ds(base, CB)], dst_v, sem2)
            d3 = pltpu.async_copy(e2_h.at[pl.ds(base, CB)], w_v, sem2)
            d1.wait()
            pltpu.sync_copy(feat_h.at[src_v], F)
            d2.wait()
            d3.wait()

            @pl.loop(0, CB // 16)
            def _rows(rb):
                r0 = rb * 16
                wvec = w_v[pl.ds(r0, 16)]
                for j in range(16):
                    wv = jnp.full((16,), wvec[j], _f32)
                    r = r0 + j
                    F[r, pl.ds(0, 16)] = F[r, pl.ds(0, 16)] * wv

            pltpu.sync_copy(F, u_s.at[dst_v], add=True)

        plsc.subcore_barrier()
        sl = pl.ds(s * _SLICE, _SLICE)
        pltpu.sync_copy(u_s.at[sl], u_o.at[c].at[sl])

    return pl.kernel(body, out_type=out_type, mesh=_mesh(),
                     scratch_types=scratch,
                     compiler_params=pltpu.CompilerParams(
                         use_tc_tiling_on_sc=False))


_fused_0 = _make_fused0()
_phase_a_1b = _make_phase_a(H=2, blend=True)
_phase_b_1 = _make_phase_b01(blend=True)
_fused_2 = _make_fused2()


def _ee_edges(emb, We, ae, H, e_feat):
    t = jnp.sum((emb @ We).reshape(8, H, 16) * ae[None], axis=-1)
    return t[e_feat].T.reshape(-1)


def _pad_nodes(arr):
    return jnp.pad(arr, ((0, _NPAD - _N),) + ((0, 0),) * (arr.ndim - 1))


def _hm_flat(arr):
    return jnp.pad(arr.T, ((0, 0), (0, _NPAD - _N))).reshape(-1)


def _rnorm(esum_parts, H):
    p = esum_parts.reshape(_NC, H, _NPAD)
    es = (p[0] + p[1])[:, :_N].T
    return jnp.where(es > 0.0, 1.0 / es, 0.0)


def kernel(x, edge_index, e_feat, fc_W, fc_b, emb0, W0, We0, al0, ar0, ae0,
           emb1, W1, We1, al1, ar1, ae1, emb2, W2, We2, al2, ar2, ae2, res2_W):
    src = edge_index[0]
    dst = edge_index[1]

    za2 = jnp.zeros((2 * _SLICE,), _f32)
    za1 = jnp.zeros((_SLICE,), _f32)
    z16 = jnp.zeros((_SLICE, 16), _f32)
    z32 = jnp.zeros((_SLICE, 32), _f32)

    h0 = x @ fc_W + fc_b
    feat0 = (h0 @ W0).reshape(_N, 2, 32)
    el0 = jnp.sum(feat0 * al0[None], -1)
    er0 = jnp.sum(feat0 * ar0[None], -1)
    feat0_hm = jnp.pad(feat0.transpose(1, 0, 2),
                       ((0, 0), (0, _NPAD - _N), (0, 0)))
    eeT0 = _ee_edges(emb0, We0, ae0, 2, e_feat)

    z32r = jnp.zeros((50048 // _NS, 32), _f32)
    u0, eexp0, esum0 = _fused_0(src, dst, _hm_flat(el0), _hm_flat(er0),
                                eeT0, z32r, za1, feat0_hm)
    es0 = esum0.reshape(_NC, _NPAD)[:, :_N].T
    rnorm0 = jnp.where(es0 > 0.0, 1.0 / es0, 0.0)

    rst0 = u0[:, :_N].transpose(1, 0, 2) * rnorm0[:, :, None]
    h1 = jax.nn.elu(rst0).reshape(_N, 64)
    feat1 = (h1 @ W1).reshape(_N, 2, 32)
    el1 = jnp.sum(feat1 * al1[None], -1)
    er1 = jnp.sum(feat1 * ar1[None], -1)
    feat1_hm = jnp.pad(feat1.transpose(1, 0, 2),
                       ((0, 0), (0, _NPAD - _N), (0, 0)))
    eeT1 = _ee_edges(emb1, We1, ae1, 2, e_feat)

    eexp1, esum1p, a0 = _phase_a_1b(src, dst, _hm_flat(el1),
                                    _hm_flat(er1), eeT1, za2,
                                    _hm_flat(rnorm0), eexp0)
    rn1_hm = _hm_flat(_rnorm(esum1p, 2))
    u1 = _phase_b_1(src, dst, eexp1, a0, rn1_hm, feat1_hm, z32)

    rst1 = u1[:, :_N].transpose(1, 0, 2) + h1.reshape(_N, 2, 32)
    h2 = jax.nn.elu(rst1).reshape(_N, 64)
    feat2 = h2 @ W2
    el2 = feat2 @ al2[0][:, None]
    er2 = feat2 @ ar2[0][:, None]
    feat2_p = _pad_nodes(feat2)
    eeT2 = _ee_edges(emb2, We2, ae2, 1, e_feat)
    resv2 = h2 @ res2_W

    u2, esum2 = _fused_2(src, dst, _hm_flat(el2), _hm_flat(er2),
                         eeT2, z16, za1, feat2_p)
    p2 = esum2.reshape(_NC, _NPAD)
    es2 = (p2[0] + p2[1])[:_N][:, None]
    rnorm2 = jnp.where(es2 > 0.0, 1.0 / es2, 0.0)

    logits = (u2[0, :_N] + u2[1, :_N]) * rnorm2 + resv2
    return logits


# speedup vs baseline: 1.0793x; 1.0544x over previous
import functools

import jax
import jax.numpy as jnp
from jax import lax
from jax.experimental import pallas as pl
from jax.experimental.pallas import tpu as pltpu
from jax.experimental.pallas import tpu_sc as plsc

_N = 50000
_E = 800000
_NC = 2
_NS = 16
_NW = _NC * _NS
_NPAD = 51200
_SLICE = _NPAD // _NS
_ALPHA = 0.05
_SLOPE = 0.2

_f32 = jnp.float32
_i32 = jnp.int32


def _mesh():
    return plsc.VectorSubcoreMesh(
        core_axis_name="c", subcore_axis_name="s", num_cores=_NC, num_subcores=_NS
    )


def _make_phase_a(H, blend, CA=3200):
    NCH = _E // CA
    NV = CA // 16
    ZL = H * _SLICE

    out_type = [
        jax.ShapeDtypeStruct((H * _E,), _f32),
        jax.ShapeDtypeStruct((_NC * H * _NPAD,), _f32),
    ]
    if blend:
        out_type.append(jax.ShapeDtypeStruct((H * _E,), _f32))

    def _hbufs(n):
        return [pltpu.VMEM((CA,), _f32) for _ in range(n)]

    scratch = (
        [pltpu.VMEM((CA,), _i32) for _ in range(2)]
        + [pltpu.VMEM((CA,), _i32) for _ in range(2 * (H - 1))]
        + _hbufs(H)
        + _hbufs(H)
        + _hbufs(H)
        + _hbufs(H)
        + [pltpu.VMEM_SHARED((H * _NPAD,), _f32)]
        + (_hbufs(3 * H) if blend else [])
        + [pltpu.SemaphoreType.DMA, pltpu.SemaphoreType.DMA]
    )

    def body(*refs):
        n_in = 8 if blend else 6
        n_out = 3 if blend else 2
        ins, outs, scr = (refs[:n_in], refs[n_in:n_in + n_out],
                          list(refs[n_in + n_out:]))
        if blend:
            src_h, dst_h, el_h, er_h, ee_h, z_h, rn0_h, e0_h = ins
            eexp_o, esum_o, a0_o = outs
        else:
            src_h, dst_h, el_h, er_h, ee_h, z_h = ins
            eexp_o, esum_o = outs

        def take(n):
            out, scr[:n] = scr[:n], []
            return out

        src_v, dst_v = take(2)
        sd1 = take(2 * (H - 1))
        srch = [src_v] + sd1[0::2]
        dsth = [dst_v] + sd1[1::2]
        elb = take(H)
        erb = take(H)
        eeb = take(H)
        exb = take(H)
        (esum_s,) = take(1)
        if blend:
            rnb = take(H)
            e0b = take(H)
            a0b = take(H)
        sem_in, sem_out = scr[:2]

        c = lax.axis_index("c")
        s = lax.axis_index("s")
        wid = s * _NC + c

        pltpu.sync_copy(z_h, esum_s.at[pl.ds(s * ZL, ZL)])
        plsc.subcore_barrier()

        @pl.loop(wid, NCH, step=_NW)
        def _chunk(k):
            base = k * CA
            ds = [pltpu.async_copy(src_h.at[pl.ds(base, CA)], src_v, sem_in),
                  pltpu.async_copy(dst_h.at[pl.ds(base, CA)], dst_v, sem_in)]
            for h in range(H):
                ds.append(pltpu.async_copy(
                    ee_h.at[pl.ds(h * _E + base, CA)], eeb[h], sem_in))
                if blend:
                    ds.append(pltpu.async_copy(
                        e0_h.at[pl.ds(h * _E + base, CA)], e0b[h], sem_in))
            for d in ds:
                d.wait()

            if H > 1:
                @pl.loop(0, NV, unroll=2)
                def _idx(i):
                    off = i * 16
                    s16 = src_v[pl.ds(off, 16)]
                    d16 = dst_v[pl.ds(off, 16)]
                    for h in range(1, H):
                        srch[h][pl.ds(off, 16)] = s16 + h * _NPAD
                        dsth[h][pl.ds(off, 16)] = d16 + h * _NPAD

            for h in range(H):
                pltpu.sync_copy(el_h.at[srch[h]], elb[h])
                pltpu.sync_copy(er_h.at[dsth[h]], erb[h])
                if blend:
                    pltpu.sync_copy(rn0_h.at[dsth[h]], rnb[h])

            @pl.loop(0, NV, unroll=2)
            def _vec(i):
                off = i * 16
                sl16 = pl.ds(off, 16)
                for h in range(H):
                    e = elb[h][sl16] + erb[h][sl16] + eeb[h][sl16]
                    e = jnp.where(e >= 0.0, e, _SLOPE * e)
                    exb[h][sl16] = jnp.exp(e)
                    if blend:
                        a0b[h][sl16] = e0b[h][sl16] * rnb[h][sl16]

            os_ = []
            for h in range(H):
                pltpu.sync_copy(exb[h], esum_s.at[dsth[h]], add=True)
                os_.append(pltpu.async_copy(
                    exb[h], eexp_o.at[pl.ds(h * _E + base, CA)], sem_out))
                if blend:
                    os_.append(pltpu.async_copy(
                        a0b[h], a0_o.at[pl.ds(h * _E + base, CA)], sem_out))
            for d in os_:
                d.wait()

        plsc.subcore_barrier()
        pltpu.sync_copy(esum_s.at[pl.ds(s * ZL, ZL)],
                        esum_o.at[pl.ds(c * H * _NPAD + s * ZL, ZL)])

    return pl.kernel(body, out_type=tuple(out_type), mesh=_mesh(),
                     scratch_types=scratch)


def _make_phase_b01(blend, CB=640, NR=_NPAD):
    NCH = _E // CB
    SLR = NR // _NS

    out_type = jax.ShapeDtypeStruct((_NC, NR, 32), _f32)
    scratch = [
        pltpu.VMEM((CB,), _i32),
        pltpu.VMEM((CB,), _i32),
        pltpu.VMEM((CB,), _f32),
        pltpu.VMEM((CB, 32), _f32),
        pltpu.VMEM_SHARED((NR, 32), _f32),
    ]
    if blend:
        scratch.append(pltpu.VMEM((CB,), _f32))
        scratch.append(pltpu.VMEM((CB,), _f32))
        scratch.append(pltpu.VMEM((CB,), _i32))
    scratch.append(pltpu.SemaphoreType.DMA)
    scratch.append(pltpu.SemaphoreType.DMA)

    def body(*refs):
        if blend:
            (src_h, dst_h, e1_h, a0_h, rn1_h, feat_h, z_h, u_o,
             src_v, dst_v, w_v, F, u_s, a0_v, rnb_v, dstc_v, sem,
             sem2) = refs
        else:
            (src_h, dst_h, e0_h, feat_h, z_h, u_o,
             src_v, dst_v, w_v, F, u_s, sem, sem2) = refs

        c = lax.axis_index("c")
        s = lax.axis_index("s")

        pltpu.sync_copy(z_h, u_s.at[pl.ds(s * SLR, SLR)])
        plsc.subcore_barrier()

        @pl.loop(s, NCH, step=_NS)
        def _chunk(k):
            base = k * CB
            di = pltpu.async_copy(src_h.at[pl.ds(base, CB)], src_v, sem)
            ds = [pltpu.async_copy(dst_h.at[pl.ds(base, CB)], dst_v, sem2)]
            if blend:
                ds.append(pltpu.async_copy(
                    e1_h.at[pl.ds(c * _E + base, CB)], w_v, sem2))
                ds.append(pltpu.async_copy(
                    a0_h.at[pl.ds(c * _E + base, CB)], a0_v, sem2))
            else:
                ds.append(pltpu.async_copy(
                    e0_h.at[pl.ds(c * _E + base, CB)], w_v, sem2))
            di.wait()
            pltpu.sync_copy(feat_h.at[c].at[src_v], F)
            for d in ds:
                d.wait()
            if blend:
                @pl.loop(0, CB // 16, unroll=2)
                def _cidx(i):
                    off = i * 16
                    dstc_v[pl.ds(off, 16)] = dst_v[pl.ds(off, 16)] + c * _NPAD

                pltpu.sync_copy(rn1_h.at[dstc_v], rnb_v)

                @pl.loop(0, CB // 16, unroll=2)
                def _wvec(i):
                    off = i * 16
                    w = ((1.0 - _ALPHA) * w_v[pl.ds(off, 16)]
                         * rnb_v[pl.ds(off, 16)]
                         + _ALPHA * a0_v[pl.ds(off, 16)])
                    w_v[pl.ds(off, 16)] = w

            @pl.loop(0, CB // 16)
            def _rows(rb):
                r0 = rb * 16
                wvec = w_v[pl.ds(r0, 16)]
                for j in range(16):
                    wv = jnp.full((16,), wvec[j], _f32)
                    r = r0 + j
                    F[r, pl.ds(0, 16)] = F[r, pl.ds(0, 16)] * wv
                    F[r, pl.ds(16, 16)] = F[r, pl.ds(16, 16)] * wv

            pltpu.sync_copy(F, u_s.at[dst_v], add=True)

        plsc.subcore_barrier()
        sl = pl.ds(s * SLR, SLR)
        pltpu.sync_copy(u_s.at[sl], u_o.at[c].at[sl])

    return pl.kernel(body, out_type=out_type, mesh=_mesh(),
                     scratch_types=scratch,
                     compiler_params=pltpu.CompilerParams(
                         use_tc_tiling_on_sc=False))


def _make_phase_b2(CB=3200):
    NCH = _E // CB

    out_type = jax.ShapeDtypeStruct((_NC, _NPAD, 16), _f32)
    scratch = [
        pltpu.VMEM((CB,), _i32),
        pltpu.VMEM((CB,), _i32),
        pltpu.VMEM((CB,), _f32),
        pltpu.VMEM((CB, 16), _f32),
        pltpu.VMEM_SHARED((_NPAD, 16), _f32),
    ]

    scratch.append(pltpu.SemaphoreType.DMA)
    scratch.append(pltpu.SemaphoreType.DMA)

    def body(src_h, dst_h, e2_h, feat_h, z_h, u_o, src_v, dst_v, w_v, F, u_s,
             sem, sem2):
        c = lax.axis_index("c")
        s = lax.axis_index("s")
        wid = s * _NC + c

        pltpu.sync_copy(z_h, u_s.at[pl.ds(s * _SLICE, _SLICE)])
        plsc.subcore_barrier()

        @pl.loop(wid, NCH, step=_NW)
        def _chunk(k):
            base = k * CB
            d1 = pltpu.async_copy(src_h.at[pl.ds(base, CB)], src_v, sem)
            d2 = pltpu.async_copy(dst_h.at[pl.ds(base, CB)], dst_v, sem2)
            d3 = pltpu.async_copy(e2_h.at[pl.ds(base, CB)], w_v, sem2)
            d1.wait()
            pltpu.sync_copy(feat_h.at[src_v], F)
            d2.wait()
            d3.wait()

            @pl.loop(0, CB // 16)
            def _rows(rb):
                r0 = rb * 16
                wvec = w_v[pl.ds(r0, 16)]
                for j in range(16):
                    wv = jnp.full((16,), wvec[j], _f32)
                    r = r0 + j
                    F[r, pl.ds(0, 16)] = F[r, pl.ds(0, 16)] * wv

            pltpu.sync_copy(F, u_s.at[dst_v], add=True)

        plsc.subcore_barrier()
        sl = pl.ds(s * _SLICE, _SLICE)
        pltpu.sync_copy(u_s.at[sl], u_o.at[c].at[sl])

    return pl.kernel(body, out_type=out_type, mesh=_mesh(),
                     scratch_types=scratch,
                     compiler_params=pltpu.CompilerParams(
                         use_tc_tiling_on_sc=False))


_phase_a_01 = _make_phase_a(H=2, blend=False)
_phase_a_1b = _make_phase_a(H=2, blend=True)
_phase_a_2 = _make_phase_a(H=1, blend=False)
_phase_b_0 = _make_phase_b01(blend=False, CB=800, NR=50048)
_phase_b_1 = _make_phase_b01(blend=True)
_phase_b_2 = _make_phase_b2()


def _ee_edges(emb, We, ae, H, e_feat):
    t = jnp.sum((emb @ We).reshape(8, H, 16) * ae[None], axis=-1)
    return t[e_feat].T.reshape(-1)


def _pad_nodes(arr):
    return jnp.pad(arr, ((0, _NPAD - _N),) + ((0, 0),) * (arr.ndim - 1))


def _hm_flat(arr):
    return jnp.pad(arr.T, ((0, 0), (0, _NPAD - _N))).reshape(-1)


def _rnorm(esum_parts, H):
    p = esum_parts.reshape(_NC, H, _NPAD)
    es = (p[0] + p[1])[:, :_N].T
    return jnp.where(es > 0.0, 1.0 / es, 0.0)


def kernel(x, edge_index, e_feat, fc_W, fc_b, emb0, W0, We0, al0, ar0, ae0,
           emb1, W1, We1, al1, ar1, ae1, emb2, W2, We2, al2, ar2, ae2, res2_W):
    src = edge_index[0]
    dst = edge_index[1]

    za2 = jnp.zeros((2 * _SLICE,), _f32)
    za1 = jnp.zeros((_SLICE,), _f32)
    z16 = jnp.zeros((_SLICE, 16), _f32)
    z32 = jnp.zeros((_SLICE, 32), _f32)

    h0 = x @ fc_W + fc_b
    feat0 = (h0 @ W0).reshape(_N, 2, 32)
    el0 = jnp.sum(feat0 * al0[None], -1)
    er0 = jnp.sum(feat0 * ar0[None], -1)
    feat0_hm = jnp.pad(feat0.transpose(1, 0, 2),
                       ((0, 0), (0, _NPAD - _N), (0, 0)))
    eeT0 = _ee_edges(emb0, We0, ae0, 2, e_feat)

    eexp0, esum0p = _phase_a_01(src, dst, _hm_flat(el0),
                                _hm_flat(er0), eeT0, za2)
    rnorm0 = _rnorm(esum0p, 2)
    z32r = jnp.zeros((50048 // _NS, 32), _f32)
    u0 = _phase_b_0(src, dst, eexp0, feat0_hm, z32r)

    rst0 = u0[:, :_N].transpose(1, 0, 2) * rnorm0[:, :, None]
    h1 = jax.nn.elu(rst0).reshape(_N, 64)
    feat1 = (h1 @ W1).reshape(_N, 2, 32)
    el1 = jnp.sum(feat1 * al1[None], -1)
    er1 = jnp.sum(feat1 * ar1[None], -1)
    feat1_hm = jnp.pad(feat1.transpose(1, 0, 2),
                       ((0, 0), (0, _NPAD - _N), (0, 0)))
    eeT1 = _ee_edges(emb1, We1, ae1, 2, e_feat)

    eexp1, esum1p, a0 = _phase_a_1b(src, dst, _hm_flat(el1),
                                    _hm_flat(er1), eeT1, za2,
                                    _hm_flat(rnorm0), eexp0)
    rn1_hm = _hm_flat(_rnorm(esum1p, 2))
    u1 = _phase_b_1(src, dst, eexp1, a0, rn1_hm, feat1_hm, z32)

    rst1 = u1[:, :_N].transpose(1, 0, 2) + h1.reshape(_N, 2, 32)
    h2 = jax.nn.elu(rst1).reshape(_N, 64)
    feat2 = h2 @ W2
    el2 = feat2 @ al2[0][:, None]
    er2 = feat2 @ ar2[0][:, None]
    feat2_p = _pad_nodes(feat2)
    eeT2 = _ee_edges(emb2, We2, ae2, 1, e_feat)
    resv2 = h2 @ res2_W

    eexp2, esum2p = _phase_a_2(src, dst, _hm_flat(el2),
                               _hm_flat(er2), eeT2, za1)
    rnorm2 = _rnorm(esum2p, 1)
    u2 = _phase_b_2(src, dst, eexp2, feat2_p, z16)

    logits = (u2[0, :_N] + u2[1, :_N]) * rnorm2 + resv2
    return logits
